# baseline (speedup 1.0000x reference)
import functools

import jax
import jax.numpy as jnp
from jax import lax
from jax.experimental import pallas as pl
from jax.experimental.pallas import tpu as pltpu
from jax.experimental.pallas import tpu_sc as plsc

F = 26
V = 100000
D = 32
B = 16384

NC, NS = 2, 16
NW = NC * NS
BPW = B // NW
DT, DS = D // 8, 8
BT, BS = B // 128, 128
WT = BPW // BS


@functools.cache
def _build():
    mesh = plsc.VectorSubcoreMesh(
        core_axis_name="c", subcore_axis_name="s", num_cores=NC, num_subcores=NS
    )
    return functools.partial(
        pl.kernel,
        out_type=jax.ShapeDtypeStruct((F, DT, BT, DS, BS), jnp.float32),
        mesh=mesh,
        scratch_types=[
            pltpu.VMEM((F, BPW), jnp.int32),
            pltpu.VMEM((WT, BS, D), jnp.float32),
            pltpu.VMEM((WT, BS, D), jnp.float32),
            pltpu.SemaphoreType.DMA,
            pltpu.SemaphoreType.DMA,
            pltpu.SemaphoreType.DMA,
            pltpu.SemaphoreType.DMA,
        ],
        compiler_params=pltpu.CompilerParams(use_tc_tiling_on_sc=False),
    )(_embed_gather)


def _embed_gather(xt_hbm, tab_hbm, out_hbm, idx_v, rows0, rows1, g0, g1, w0, w1):
    wid = lax.axis_index("s") * NC + lax.axis_index("c")
    b0 = wid * BPW

    pltpu.sync_copy(xt_hbm.at[:, pl.ds(b0, BPW)], idx_v)

    def add_off(j, carry):
        f = j // (BPW // 16)
        l = j - f * (BPW // 16)
        sl = (f, pl.ds(l * 16, 16))
        idx_v[sl] = idx_v[sl] + f * V
        return carry

    lax.fori_loop(0, F * (BPW // 16), add_off, 0)

    rows = (rows0, rows1)
    gsem = (g0, g1)
    wsem = (w0, w1)

    def fire_gather(f, nb):
        for bt in range(WT):
            pltpu.async_copy(
                tab_hbm.at[idx_v.at[f, pl.ds(bt * BS, BS)]],
                rows[nb].at[bt],
                gsem[nb],
            )

    def wait_gather(f, nb):
        for bt in range(WT):
            pltpu.make_async_copy(
                tab_hbm.at[idx_v.at[f, pl.ds(bt * BS, BS)]],
                rows[nb].at[bt],
                gsem[nb],
            ).wait()

    def _write_parts(f, nb):
        for dt in range(DT):
            for ds in range(DS):
                yield (
                    rows[nb].at[:, :, dt * 8 + ds],
                    out_hbm.at[f, dt, pl.ds(wid * WT, WT), ds, :],
                )

    def fire_write(f, nb):
        for src, dst in _write_parts(f, nb):
            pltpu.async_copy(src, dst, wsem[nb])

    def wait_write(f, nb):
        for src, dst in _write_parts(f, nb):
            pltpu.make_async_copy(src, dst, wsem[nb]).wait()

    fire_gather(0, 0)

    def step(c, nb, first=False):
        if first:
            @pl.when(c >= 1)
            def _():
                wait_write(c - 1, 1 - nb)
        else:
            wait_write(c - 1, 1 - nb)
        fire_gather(c + 1, 1 - nb)
        wait_gather(c, nb)
        fire_write(c, nb)

    def pair(k, carry):
        c = k * 2
        step(c, 0, first=True)
        step(c + 1, 1)
        return carry

    lax.fori_loop(0, (F - 2) // 2, pair, 0)
    step(F - 2, 0)
    wait_gather(F - 1, 1)
    fire_write(F - 1, 1)
    wait_write(F - 2, 0)
    wait_write(F - 1, 1)


def kernel(x_sparse, tables):
    xt = jnp.transpose(x_sparse.astype(jnp.int32))
    tab4 = lax.optimization_barrier(tables.reshape(F * V // 4, 4 * D))
    tab_flat = tab4.reshape(F * V, D)
    out5 = _build()(xt, tab_flat)
    out = jnp.transpose(out5, (2, 4, 0, 1, 3))
    return out.reshape(B, F, D)


# speedup vs baseline: 39.5235x; 39.5235x over previous
import functools

import jax
import jax.numpy as jnp
from jax import lax
from jax.experimental import pallas as pl
from jax.experimental.pallas import tpu as pltpu
from jax.experimental.pallas import tpu_sc as plsc

F = 26
V = 100000
D = 32
B = 16384

NC, NS = 2, 16
NW = NC * NS
BPW = B // NW
NG = 7


@functools.cache
def _build():
    mesh = plsc.VectorSubcoreMesh(
        core_axis_name="c", subcore_axis_name="s", num_cores=NC, num_subcores=NS
    )
    return functools.partial(
        pl.kernel,
        out_type=jax.ShapeDtypeStruct((F, B, D), jnp.float32),
        mesh=mesh,
        scratch_types=[
            pltpu.VMEM((F, BPW), jnp.int32),
            pltpu.VMEM((BPW, D), jnp.float32),
            pltpu.VMEM((BPW, D), jnp.float32),
            pltpu.SemaphoreType.DMA,
            pltpu.SemaphoreType.DMA,
            pltpu.SemaphoreType.DMA,
            pltpu.SemaphoreType.DMA,
        ],
        compiler_params=pltpu.CompilerParams(use_tc_tiling_on_sc=False),
    )(_embed_gather)


def _embed_gather(xt_hbm, *args):
    tabs = args[:NG]
    out_hbm = args[NG]
    idx_v, rows0, rows1, g0, g1, w0, w1 = args[NG + 1:]

    wid = lax.axis_index("s") * NC + lax.axis_index("c")
    b0 = wid * BPW

    pltpu.sync_copy(xt_hbm.at[:, pl.ds(b0, BPW)], idx_v)

    def add_off(j, carry):
        f = j // (BPW // 16)
        l = j - f * (BPW // 16)
        sl = (f, pl.ds(l * 16, 16))
        idx_v[sl] = idx_v[sl] * 4 + lax.rem(f, 4)
        return carry

    lax.fori_loop(0, F * (BPW // 16), add_off, 0)

    rows = (rows0, rows1)
    gsem = (g0, g1)
    wsem = (w0, w1)

    def fire_gather(f, nb):
        pltpu.async_copy(tabs[f // 4].at[idx_v.at[f]], rows[nb], gsem[nb])

    def wait_gather(f, nb):
        pltpu.make_async_copy(
            tabs[f // 4].at[idx_v.at[f]], rows[nb], gsem[nb]
        ).wait()

    def fire_write(f, nb):
        pltpu.async_copy(rows[nb], out_hbm.at[f, pl.ds(b0, BPW), :], wsem[nb])

    def wait_write(f, nb):
        pltpu.make_async_copy(
            rows[nb], out_hbm.at[f, pl.ds(b0, BPW), :], wsem[nb]
        ).wait()

    fire_gather(0, 0)
    for f in range(F - 1):
        nb = f % 2
        if f >= 1:
            wait_write(f - 1, 1 - nb)
        fire_gather(f + 1, 1 - nb)
        wait_gather(f, nb)
        fire_write(f, nb)
    last = F - 1
    wait_gather(last, last % 2)
    fire_write(last, last % 2)
    wait_write(last - 1, (last - 1) % 2)
    wait_write(last, last % 2)


def kernel(x_sparse, tables):
    xt = jnp.transpose(x_sparse.astype(jnp.int32))
    tt = jnp.transpose(tables, (0, 2, 1)).reshape(F * D, V)
    groups = []
    for g in range(NG - 1):
        tg = jnp.transpose(tt[g * 128:(g + 1) * 128, :])
        groups.append(tg.reshape(4 * V, D))
    tail = jnp.pad(tt[(NG - 1) * 128:, :], ((0, 64), (0, 0)))
    groups.append(jnp.transpose(tail).reshape(4 * V, D))
    out_fmajor = _build()(xt, *groups)
    return jnp.transpose(out_fmajor, (1, 0, 2))
